# baseline (speedup 1.0000x reference)
import functools

import jax
import jax.numpy as jnp
from jax import lax
from jax.experimental import pallas as pl
from jax.experimental.pallas import tpu as pltpu
from jax.experimental.pallas import tpu_sc as plsc

BLK = 256
NC, NS = 2, 16
NW = NC * NS
CH = 16


def _router_meta_body(x_ref, wr_ref, pos_ref, w01_ref, be_ref, aux_ref,
                      *, n_tokens, n_experts, nb_tot):
    x = x_ref[...]
    logits = lax.dot_general(
        x, wr_ref[...], (((1,), (1,)), ((), ())),
        preferred_element_type=jnp.float32)

    e_iota = lax.broadcasted_iota(jnp.int32, logits.shape, 1)
    m1 = jnp.max(logits, axis=-1, keepdims=True)
    i1 = jnp.min(jnp.where(logits == m1, e_iota, n_experts), axis=-1,
                 keepdims=True)
    oh1 = (e_iota == i1).astype(jnp.float32)
    masked = jnp.where(e_iota == i1, -jnp.inf, logits)
    m2 = jnp.max(masked, axis=-1, keepdims=True)
    i2 = jnp.min(jnp.where(masked == m2, e_iota, n_experts), axis=-1,
                 keepdims=True)
    oh2 = (e_iota == i2).astype(jnp.float32)
    w2 = 1.0 / (1.0 + jnp.exp(m1 - m2))
    w1 = 1.0 - w2
    w01_ref[...] = jnp.concatenate([w1, w2], axis=1)

    cnt = oh1 + oh2
    c = cnt
    s = 1
    while s < n_tokens:
        c = c + jnp.concatenate(
            [jnp.zeros((s, n_experts), jnp.float32), c[:-s, :]], axis=0)
        s *= 2
    c_excl = c - cnt
    counts = c[n_tokens - 1:n_tokens, :]
    ci = counts.astype(jnp.int32)
    ca = ((ci + (BLK - 1)) // BLK) * BLK
    off = ca
    s = 1
    while s < n_experts:
        off = off + jnp.concatenate(
            [jnp.zeros((1, s), jnp.int32), off[:, :-s]], axis=1)
        s *= 2
    off_excl_f = (off - ca).astype(jnp.float32)

    slot = off_excl_f + c_excl
    p0 = jnp.sum(oh1 * slot, axis=1, keepdims=True)
    p1 = jnp.sum(oh2 * slot, axis=1, keepdims=True)
    pos_ref[...] = jnp.concatenate([p0, p1], axis=1).astype(jnp.int32)

    bstart = lax.broadcasted_iota(jnp.int32, (nb_tot, n_experts), 0) * BLK
    be_ref[...] = jnp.sum(
        (jnp.broadcast_to(off, (nb_tot, n_experts)) <= bstart
         ).astype(jnp.int32), axis=1, keepdims=True)

    z = jnp.exp(logits - m1)
    sc = z / jnp.sum(z, axis=-1, keepdims=True)
    pi_sum = jnp.sum(sc, axis=0, keepdims=True)
    aux_ref[...] = (jnp.sum(pi_sum * counts)
                    / float(n_tokens * n_tokens)).reshape(1, 1)


def _gmm_body(be_ref, xp_ref, w_ref, y_ref, *, n_experts):
    @pl.when(be_ref[pl.program_id(0)] < n_experts)
    def _():
        y_ref[...] = lax.dot_general(
            xp_ref[...].astype(jnp.bfloat16), w_ref[0],
            (((1,), (1,)), ((), ())), preferred_element_type=jnp.float32)


def _shared_body(x_ref, ws_ref, o_ref):
    xb = x_ref[...].astype(jnp.bfloat16)
    ws = (ws_ref[0].astype(jnp.float32)
          + ws_ref[1].astype(jnp.float32)).astype(jnp.bfloat16)
    o_ref[...] = lax.dot_general(xb, ws, (((1,), (1,)), ((), ())),
                                 preferred_element_type=jnp.float32)


def _final_body(sh_ref, y0_ref, y1_ref, w01_ref, out_ref):
    w01 = w01_ref[...]
    out_ref[...] = (sh_ref[...] + w01[:, 0:1] * y0_ref[...]
                    + w01[:, 1:2] * y1_ref[...])


def _make_dispatch(n_tokens, d, nk_pad):
    tpw = n_tokens // NW
    nch = tpw // CH
    mesh = plsc.VectorSubcoreMesh(core_axis_name="c", subcore_axis_name="s")

    @functools.partial(
        pl.kernel, mesh=mesh,
        out_type=jax.ShapeDtypeStruct((nk_pad, d), jnp.float32),
        scratch_types=[
            pltpu.VMEM((nch, 2, CH), jnp.int32),
            pltpu.VMEM((CH, d), jnp.float32),
            pltpu.VMEM((CH, d), jnp.float32),
        ] + [pltpu.SemaphoreType.DMA] * 6,
    )
    def dispatch(x_hbm, pos_hbm, xp_hbm, idx_v, buf0, buf1,
                 sr0, sr1, sw00, sw01, sw10, sw11):
        wid = lax.axis_index("s") * NC + lax.axis_index("c")
        base = wid * tpw
        pltpu.sync_copy(pos_hbm.at[wid], idx_v)
        bufs = (buf0, buf1)
        sr = (sr0, sr1)
        sw = ((sw00, sw01), (sw10, sw11))
        reads = [None] * nch
        writes = [None] * nch
        reads[0] = pltpu.async_copy(x_hbm.at[pl.ds(base, CH)], bufs[0], sr[0])
        for c in range(nch):
            b = c % 2
            reads[c].wait()
            if c + 1 < nch:
                if c >= 1:
                    writes[c - 1][0].wait()
                    writes[c - 1][1].wait()
                reads[c + 1] = pltpu.async_copy(
                    x_hbm.at[pl.ds(base + (c + 1) * CH, CH)],
                    bufs[1 - b], sr[1 - b])
            writes[c] = (
                pltpu.async_copy(bufs[b], xp_hbm.at[idx_v.at[c, 0]], sw[b][0]),
                pltpu.async_copy(bufs[b], xp_hbm.at[idx_v.at[c, 1]], sw[b][1]))
        writes[nch - 1][0].wait()
        writes[nch - 1][1].wait()
        if nch >= 2:
            writes[nch - 2][0].wait()
            writes[nch - 2][1].wait()

    return dispatch


def _make_combine(n_tokens, d, nk_pad):
    tpw = n_tokens // NW
    nch = tpw // CH
    mesh = plsc.VectorSubcoreMesh(core_axis_name="c", subcore_axis_name="s")

    @functools.partial(
        pl.kernel, mesh=mesh,
        out_type=(jax.ShapeDtypeStruct((n_tokens, d), jnp.float32),
                  jax.ShapeDtypeStruct((n_tokens, d), jnp.float32)),
        scratch_types=[
            pltpu.VMEM((nch, 2 * CH), jnp.int32),
            pltpu.VMEM((2 * CH, d), jnp.float32),
            pltpu.VMEM((2 * CH, d), jnp.float32),
        ] + [pltpu.SemaphoreType.DMA] * 6,
    )
    def combine(y_hbm, pos_hbm, y0_hbm, y1_hbm, idx_v, gbuf0, gbuf1,
                sg0, sg1, sw00, sw01, sw10, sw11):
        wid = lax.axis_index("s") * NC + lax.axis_index("c")
        base = wid * tpw
        pltpu.sync_copy(pos_hbm.at[wid], idx_v)
        gbufs = (gbuf0, gbuf1)
        sg = (sg0, sg1)
        sw = ((sw00, sw01), (sw10, sw11))
        reads = [None] * nch
        writes = [None] * nch
        reads[0] = pltpu.async_copy(y_hbm.at[idx_v.at[0]], gbufs[0], sg[0])
        for c in range(nch):
            b = c % 2
            reads[c].wait()
            if c + 1 < nch:
                if c >= 1:
                    writes[c - 1][0].wait()
                    writes[c - 1][1].wait()
                reads[c + 1] = pltpu.async_copy(
                    y_hbm.at[idx_v.at[c + 1]], gbufs[1 - b], sg[1 - b])
            dst = pl.ds(base + c * CH, CH)
            writes[c] = (
                pltpu.async_copy(gbufs[b].at[pl.ds(0, CH)],
                                 y0_hbm.at[dst], sw[b][0]),
                pltpu.async_copy(gbufs[b].at[pl.ds(CH, CH)],
                                 y1_hbm.at[dst], sw[b][1]))
        writes[nch - 1][0].wait()
        writes[nch - 1][1].wait()
        if nch >= 2:
            writes[nch - 2][0].wait()
            writes[nch - 2][1].wait()

    return combine


def _router_meta(x, W_router, nb_tot):
    n_tokens, _ = x.shape
    n_experts = W_router.shape[0]
    return pl.pallas_call(
        functools.partial(_router_meta_body, n_tokens=n_tokens,
                          n_experts=n_experts, nb_tot=nb_tot),
        out_shape=[
            jax.ShapeDtypeStruct((n_tokens, 2), jnp.int32),
            jax.ShapeDtypeStruct((n_tokens, 2), jnp.float32),
            jax.ShapeDtypeStruct((nb_tot, 1), jnp.int32),
            jax.ShapeDtypeStruct((1, 1), jnp.float32),
        ],
    )(x, W_router)


def _gmm(be, x_perm, we, n_experts, d):
    nb_tot = be.shape[0]
    grid_spec = pltpu.PrefetchScalarGridSpec(
        num_scalar_prefetch=1,
        grid=(nb_tot,),
        in_specs=[
            pl.BlockSpec((BLK, d), lambda i, be_r: (i, 0)),
            pl.BlockSpec((1, d, d),
                         lambda i, be_r: (jnp.minimum(be_r[i], n_experts - 1),
                                          0, 0)),
        ],
        out_specs=pl.BlockSpec((BLK, d), lambda i, be_r: (i, 0)),
    )
    return pl.pallas_call(
        functools.partial(_gmm_body, n_experts=n_experts),
        grid_spec=grid_spec,
        out_shape=jax.ShapeDtypeStruct((x_perm.shape[0], d), jnp.float32),
    )(be, x_perm, we)


def _shared(x, ws):
    n_tokens, d = x.shape
    blk = 512
    return pl.pallas_call(
        _shared_body,
        grid=(n_tokens // blk,),
        in_specs=[
            pl.BlockSpec((blk, d), lambda i: (i, 0)),
            pl.BlockSpec((2, d, d), lambda i: (0, 0, 0)),
        ],
        out_specs=pl.BlockSpec((blk, d), lambda i: (i, 0)),
        out_shape=jax.ShapeDtypeStruct((n_tokens, d), jnp.float32),
    )(x, ws)


def _final(sh, y0, y1, w01):
    n_tokens, d = sh.shape
    blk = 512
    return pl.pallas_call(
        _final_body,
        grid=(n_tokens // blk,),
        in_specs=[
            pl.BlockSpec((blk, d), lambda i: (i, 0)),
            pl.BlockSpec((blk, d), lambda i: (i, 0)),
            pl.BlockSpec((blk, d), lambda i: (i, 0)),
            pl.BlockSpec((blk, 2), lambda i: (i, 0)),
        ],
        out_specs=pl.BlockSpec((blk, d), lambda i: (i, 0)),
        out_shape=jax.ShapeDtypeStruct((n_tokens, d), jnp.float32),
    )(sh, y0, y1, w01)


def kernel(feat, W_router, W_shared, W_experts):
    B, S, d = feat.shape
    N = B * S
    E = W_router.shape[0]
    topk = 2
    nb_tot = (N * topk) // BLK + E
    nk_pad = nb_tot * BLK

    x = feat.reshape(N, d)
    we = W_experts.astype(jnp.bfloat16)
    ws = W_shared.reshape(-1, d, d).astype(jnp.bfloat16)

    pos, w01, be2d, aux = _router_meta(x, W_router, nb_tot)
    tpw = N // NW
    nch = tpw // CH
    pos_sc = pos.reshape(NW, nch, CH, 2).transpose(0, 1, 3, 2)
    pos_disp = pos_sc
    pos_comb = pos_sc.reshape(NW, nch, 2 * CH)
    be = be2d.reshape(nb_tot)

    x_perm = _make_dispatch(N, d, nk_pad)(x, pos_disp)
    sh = _shared(x, ws)
    y = _gmm(be, x_perm, we, E, d)
    y0, y1 = _make_combine(N, d, nk_pad)(y, pos_comb)
    out = _final(sh, y0, y1, w01)
    return out.reshape(B, S, d), aux[0, 0]


# speedup vs baseline: 1.4190x; 1.4190x over previous
import functools

import jax
import jax.numpy as jnp
from jax.experimental import pallas as pl
from jax.experimental.pallas import tpu as pltpu


def _moe_dense_body(x_ref, wr_ref, we_ref, ws_ref, out_ref, aux_ref, acc_ref,
                    *, n_tokens: int, n_experts: int):
    i = pl.program_id(0)
    nb = pl.num_programs(0)
    x = x_ref[...]

    logits = jax.lax.dot_general(
        x, wr_ref[...], (((1,), (1,)), ((), ())),
        preferred_element_type=jnp.float32)

    e_iota = jax.lax.broadcasted_iota(jnp.int32, logits.shape, 1)
    m1 = jnp.max(logits, axis=-1, keepdims=True)
    i1 = jnp.min(jnp.where(logits == m1, e_iota, n_experts), axis=-1,
                 keepdims=True)
    oh1 = (e_iota == i1).astype(jnp.float32)
    masked = jnp.where(e_iota == i1, -jnp.inf, logits)
    m2 = jnp.max(masked, axis=-1, keepdims=True)
    i2 = jnp.min(jnp.where(masked == m2, e_iota, n_experts), axis=-1,
                 keepdims=True)
    oh2 = (e_iota == i2).astype(jnp.float32)
    w2 = 1.0 / (1.0 + jnp.exp(m1 - m2))
    w1 = 1.0 - w2
    gate = w1 * oh1 + w2 * oh2

    z = jnp.exp(logits - m1)
    sc = z / jnp.sum(z, axis=-1, keepdims=True)

    @pl.when(i == 0)
    def _init():
        acc_ref[...] = jnp.zeros_like(acc_ref)

    acc_ref[0, :] += jnp.sum(sc, axis=0)
    acc_ref[1, :] += jnp.sum(oh1 + oh2, axis=0)

    xb = x.astype(jnp.bfloat16)
    acc = jnp.zeros(out_ref.shape, jnp.float32)
    for e in range(n_experts):
        ye = jax.lax.dot_general(
            xb, we_ref[e], (((1,), (0,)), ((), ())),
            preferred_element_type=jnp.float32)
        acc += gate[:, e:e + 1] * ye
    ws = (ws_ref[0].astype(jnp.float32)
          + ws_ref[1].astype(jnp.float32)).astype(jnp.bfloat16)
    acc += jax.lax.dot_general(
        xb, ws, (((1,), (0,)), ((), ())), preferred_element_type=jnp.float32)
    out_ref[...] = acc

    @pl.when(i == nb - 1)
    def _fin():
        pi = acc_ref[0, :] / n_tokens
        fi = acc_ref[1, :] / n_tokens
        aux_ref[...] = jnp.sum(pi * fi).reshape(1, 1)


def kernel(feat, W_router, W_shared, W_experts):
    B, S, d = feat.shape
    N = B * S
    E = W_router.shape[0]
    x = feat.reshape(N, d)
    we = W_experts.astype(jnp.bfloat16).transpose(0, 2, 1)
    ws = W_shared.reshape(-1, d, d).astype(jnp.bfloat16).transpose(0, 2, 1)
    n_shared = ws.shape[0]
    assert n_shared == 2
    BLK = 512
    nb = N // BLK
    out, aux = pl.pallas_call(
        functools.partial(_moe_dense_body, n_tokens=N, n_experts=E),
        grid=(nb,),
        in_specs=[
            pl.BlockSpec((BLK, d), lambda i: (i, 0)),
            pl.BlockSpec((E, d), lambda i: (0, 0)),
            pl.BlockSpec((E, d, d), lambda i: (0, 0, 0)),
            pl.BlockSpec((n_shared, d, d), lambda i: (0, 0, 0)),
        ],
        out_specs=[
            pl.BlockSpec((BLK, d), lambda i: (i, 0)),
            pl.BlockSpec((1, 1), lambda i: (0, 0)),
        ],
        out_shape=[
            jax.ShapeDtypeStruct((N, d), jnp.float32),
            jax.ShapeDtypeStruct((1, 1), jnp.float32),
        ],
        scratch_shapes=[pltpu.VMEM((2, E), jnp.float32)],
    )(x, W_router, we, ws)
    return out.reshape(B, S, d), aux[0, 0]


# speedup vs baseline: 1.5432x; 1.0875x over previous
import functools

import jax
import jax.numpy as jnp
from jax.experimental import pallas as pl
from jax.experimental.pallas import tpu as pltpu


def _moe_dense_body(x_ref, wr_ref, we_ref, ws_ref, out_ref, aux_ref, acc_ref,
                    *, n_tokens: int, n_experts: int):
    i = pl.program_id(0)
    nb = pl.num_programs(0)
    x = x_ref[...]

    logits = jax.lax.dot_general(
        x, wr_ref[...], (((1,), (1,)), ((), ())),
        preferred_element_type=jnp.float32)

    e_iota = jax.lax.broadcasted_iota(jnp.int32, logits.shape, 1)
    m1 = jnp.max(logits, axis=-1, keepdims=True)
    i1 = jnp.min(jnp.where(logits == m1, e_iota, n_experts), axis=-1,
                 keepdims=True)
    oh1 = (e_iota == i1).astype(jnp.float32)
    masked = jnp.where(e_iota == i1, -jnp.inf, logits)
    m2 = jnp.max(masked, axis=-1, keepdims=True)
    i2 = jnp.min(jnp.where(masked == m2, e_iota, n_experts), axis=-1,
                 keepdims=True)
    oh2 = (e_iota == i2).astype(jnp.float32)
    w2 = 1.0 / (1.0 + jnp.exp(m1 - m2))
    w1 = 1.0 - w2
    gate = w1 * oh1 + w2 * oh2

    z = jnp.exp(logits - m1)
    sc = z / jnp.sum(z, axis=-1, keepdims=True)

    @pl.when(i == 0)
    def _init():
        acc_ref[...] = jnp.zeros_like(acc_ref)

    acc_ref[0, :] += jnp.sum(sc, axis=0)
    acc_ref[1, :] += jnp.sum(oh1 + oh2, axis=0)

    xb = x.astype(jnp.bfloat16)
    d = out_ref.shape[1]
    y_all = jax.lax.dot_general(
        xb, we_ref[...], (((1,), (1,)), ((), ())),
        preferred_element_type=jnp.float32)
    acc = jnp.zeros(out_ref.shape, jnp.float32)
    for e in range(n_experts):
        acc += gate[:, e:e + 1] * y_all[:, e * d:(e + 1) * d]
    ws = (ws_ref[0].astype(jnp.float32)
          + ws_ref[1].astype(jnp.float32)).astype(jnp.bfloat16)
    acc += jax.lax.dot_general(
        xb, ws, (((1,), (1,)), ((), ())), preferred_element_type=jnp.float32)
    out_ref[...] = acc

    @pl.when(i == nb - 1)
    def _fin():
        pi = acc_ref[0, :] / n_tokens
        fi = acc_ref[1, :] / n_tokens
        aux_ref[...] = jnp.sum(pi * fi).reshape(1, 1)


def kernel(feat, W_router, W_shared, W_experts):
    B, S, d = feat.shape
    N = B * S
    E = W_router.shape[0]
    x = feat.reshape(N, d)
    we = W_experts.reshape(E * d, d).astype(jnp.bfloat16)
    ws = W_shared.reshape(-1, d, d).astype(jnp.bfloat16)
    n_shared = ws.shape[0]
    assert n_shared == 2
    BLK = 512
    nb = N // BLK
    out, aux = pl.pallas_call(
        functools.partial(_moe_dense_body, n_tokens=N, n_experts=E),
        grid=(nb,),
        in_specs=[
            pl.BlockSpec((BLK, d), lambda i: (i, 0)),
            pl.BlockSpec((E, d), lambda i: (0, 0)),
            pl.BlockSpec((E * d, d), lambda i: (0, 0)),
            pl.BlockSpec((n_shared, d, d), lambda i: (0, 0, 0)),
        ],
        out_specs=[
            pl.BlockSpec((BLK, d), lambda i: (i, 0)),
            pl.BlockSpec((1, 1), lambda i: (0, 0)),
        ],
        out_shape=[
            jax.ShapeDtypeStruct((N, d), jnp.float32),
            jax.ShapeDtypeStruct((1, 1), jnp.float32),
        ],
        scratch_shapes=[pltpu.VMEM((2, E), jnp.float32)],
    )(x, W_router, we, ws)
    return out.reshape(B, S, d), aux[0, 0]
